# Initial kernel scaffold; baseline (speedup 1.0000x reference)
import jax
import jax.numpy as jnp
from jax.experimental import pallas as pl


def kernel(xyz1, xyz2, points1, points2, feat1, feat2, W0, b0, g0, be0, W1, b1, g1, be1):
    raise NotImplementedError("write your pallas kernel here")


# speedup vs baseline: 13.5477x; 13.5477x over previous
import functools

import jax
import jax.numpy as jnp
from jax.experimental import pallas as pl


def _k1_body(q_ref, k_ref, p2_ref, p1_ref, w0p_ref, w0i_ref,
             z1_ref, sums_ref, *, S, NB):
    q = q_ref[0]
    k = k_ref[0]
    dot = jax.lax.dot_general(k, q, (((1,), (1,)), ((), ())),
                              preferred_element_type=jnp.float32)
    kn = jnp.sum(k * k, axis=1, keepdims=True)
    ones8 = jnp.ones((8, q.shape[1]), jnp.float32)
    qn8 = jax.lax.dot_general(ones8, q * q, (((1,), (1,)), ((), ())),
                              preferred_element_type=jnp.float32)
    dist = kn + qn8[0:1, :] - 2.0 * dot

    ii = jax.lax.broadcasted_iota(jnp.int32, (S, NB), 0)
    d = dist
    ms = []
    ams = []
    for _ in range(3):
        m = jnp.min(d, axis=0, keepdims=True)
        am = jnp.min(jnp.where(d == m, ii, S), axis=0, keepdims=True)
        ms.append(m)
        ams.append(am)
        d = jnp.where(ii == am, jnp.float32(jnp.inf), d)

    r0 = 1.0 / (ms[0] + 1e-8)
    r1 = 1.0 / (ms[1] + 1e-8)
    r2 = 1.0 / (ms[2] + 1e-8)
    rnorm = r0 + r1 + r2
    w0 = r0 / rnorm
    w1 = r1 / rnorm
    w2 = r2 / rnorm

    zero = jnp.zeros((S, NB), jnp.float32)
    wdT = (jnp.where(ii == ams[0], w0, zero)
           + jnp.where(ii == ams[1], w1, zero)
           + jnp.where(ii == ams[2], w2, zero))

    interpT = jax.lax.dot_general(p2_ref[0], wdT, (((1,), (0,)), ((), ())),
                                  preferred_element_type=jnp.float32)
    z1 = (jax.lax.dot_general(w0i_ref[...], interpT, (((1,), (0,)), ((), ())),
                              preferred_element_type=jnp.float32)
          + jax.lax.dot_general(w0p_ref[...], p1_ref[0], (((1,), (0,)), ((), ())),
                                preferred_element_type=jnp.float32))
    z1_ref[0] = z1

    part = jnp.concatenate(
        [jnp.sum(z1, axis=1, keepdims=True),
         jnp.sum(z1 * z1, axis=1, keepdims=True)], axis=1)
    first = (pl.program_id(0) == 0) & (pl.program_id(1) == 0)

    @pl.when(first)
    def _():
        sums_ref[...] = part

    @pl.when(jnp.logical_not(first))
    def _():
        sums_ref[...] = sums_ref[...] + part


def _k2_body(z1_ref, sums_ref, g_ref, be_ref, w1_ref, z2_ref, sums2_ref,
             *, count):
    s = sums_ref[:, 0:1]
    sq = sums_ref[:, 1:2]
    mu = s / count
    var = sq / count - mu * mu
    inv = g_ref[...] * jax.lax.rsqrt(var + 1e-5)
    shift = be_ref[...] - mu * inv
    h = jnp.maximum(z1_ref[0] * inv + shift, 0.0)
    z2 = jax.lax.dot_general(w1_ref[...], h, (((1,), (0,)), ((), ())),
                             preferred_element_type=jnp.float32)
    z2_ref[0] = z2

    part = jnp.concatenate(
        [jnp.sum(z2, axis=1, keepdims=True),
         jnp.sum(z2 * z2, axis=1, keepdims=True)], axis=1)
    first = (pl.program_id(0) == 0) & (pl.program_id(1) == 0)

    @pl.when(first)
    def _():
        sums2_ref[...] = part

    @pl.when(jnp.logical_not(first))
    def _():
        sums2_ref[...] = sums2_ref[...] + part


def _k3_body(z2_ref, sums_ref, g_ref, be_ref, y_ref, *, count):
    s = sums_ref[:, 0:1]
    sq = sums_ref[:, 1:2]
    mu = s / count
    var = sq / count - mu * mu
    inv = g_ref[...] * jax.lax.rsqrt(var + 1e-5)
    shift = be_ref[...] - mu * inv
    y_ref[0] = jnp.maximum(z2_ref[0] * inv + shift, 0.0)


def kernel(xyz1, xyz2, points1, points2, feat1, feat2,
           W0, b0, g0, be0, W1, b1, g1, be1):
    B, _, N = xyz1.shape
    S = xyz2.shape[2]
    D1 = points1.shape[1]
    D2 = points2.shape[1]
    CF = feat1.shape[2]
    C1 = W0.shape[0]
    C2 = W1.shape[0]
    NB = min(256, N)
    NJ = N // NB
    C = 3 + CF
    CP = ((C + 7) // 8) * 8

    x1t = jnp.transpose(xyz1, (0, 2, 1))
    x2t = jnp.transpose(xyz2, (0, 2, 1))
    q = jnp.concatenate(
        [x1t, feat1, jnp.zeros((B, N, CP - C), jnp.float32)], axis=2)
    kq = jnp.concatenate(
        [x2t, feat2, jnp.zeros((B, S, CP - C), jnp.float32)], axis=2)
    w0p = W0[:, :D1]
    w0i = W0[:, D1:]
    g0c = g0[:, None]
    be0c = be0[:, None]
    g1c = g1[:, None]
    be1c = be1[:, None]

    z1, sums1 = pl.pallas_call(
        functools.partial(_k1_body, S=S, NB=NB),
        grid=(B, NJ),
        in_specs=[
            pl.BlockSpec((1, NB, CP), lambda b, j: (b, j, 0)),
            pl.BlockSpec((1, S, CP), lambda b, j: (b, 0, 0)),
            pl.BlockSpec((1, D2, S), lambda b, j: (b, 0, 0)),
            pl.BlockSpec((1, D1, NB), lambda b, j: (b, 0, j)),
            pl.BlockSpec((C1, D1), lambda b, j: (0, 0)),
            pl.BlockSpec((C1, D2), lambda b, j: (0, 0)),
        ],
        out_specs=[
            pl.BlockSpec((1, C1, NB), lambda b, j: (b, 0, j)),
            pl.BlockSpec((C1, 2), lambda b, j: (0, 0)),
        ],
        out_shape=[
            jax.ShapeDtypeStruct((B, C1, N), jnp.float32),
            jax.ShapeDtypeStruct((C1, 2), jnp.float32),
        ],
    )(q, kq, points2, points1, w0p, w0i)

    count = float(B * N)
    z2, sums2 = pl.pallas_call(
        functools.partial(_k2_body, count=count),
        grid=(B, NJ),
        in_specs=[
            pl.BlockSpec((1, C1, NB), lambda b, j: (b, 0, j)),
            pl.BlockSpec((C1, 2), lambda b, j: (0, 0)),
            pl.BlockSpec((C1, 1), lambda b, j: (0, 0)),
            pl.BlockSpec((C1, 1), lambda b, j: (0, 0)),
            pl.BlockSpec((C2, C1), lambda b, j: (0, 0)),
        ],
        out_specs=[
            pl.BlockSpec((1, C2, NB), lambda b, j: (b, 0, j)),
            pl.BlockSpec((C2, 2), lambda b, j: (0, 0)),
        ],
        out_shape=[
            jax.ShapeDtypeStruct((B, C2, N), jnp.float32),
            jax.ShapeDtypeStruct((C2, 2), jnp.float32),
        ],
    )(z1, sums1, g0c, be0c, W1)

    y = pl.pallas_call(
        functools.partial(_k3_body, count=count),
        grid=(B, NJ),
        in_specs=[
            pl.BlockSpec((1, C2, NB), lambda b, j: (b, 0, j)),
            pl.BlockSpec((C2, 2), lambda b, j: (0, 0)),
            pl.BlockSpec((C2, 1), lambda b, j: (0, 0)),
            pl.BlockSpec((C2, 1), lambda b, j: (0, 0)),
        ],
        out_specs=pl.BlockSpec((1, C2, NB), lambda b, j: (b, 0, j)),
        out_shape=jax.ShapeDtypeStruct((B, C2, N), jnp.float32),
    )(z2, sums2, g1c, be1c)

    return y


# speedup vs baseline: 15.6088x; 1.1521x over previous
import functools

import jax
import jax.numpy as jnp
from jax.experimental import pallas as pl


def _k1_body(q_ref, k_ref, p2_ref, p1_ref, w0p_ref, w0i_ref,
             z1_ref, sums_ref, *, S, NB):
    q = q_ref[0]
    k = k_ref[0]
    dot = jax.lax.dot_general(k, q, (((1,), (1,)), ((), ())),
                              preferred_element_type=jnp.float32)
    kn = jnp.sum(k * k, axis=1, keepdims=True)
    ones8 = jnp.ones((8, q.shape[1]), jnp.float32)
    qn8 = jax.lax.dot_general(ones8, q * q, (((1,), (1,)), ((), ())),
                              preferred_element_type=jnp.float32)
    dist = kn + qn8[0:1, :] - 2.0 * dot

    inf = jnp.float32(jnp.inf)
    m0 = jnp.min(dist, axis=0, keepdims=True)
    c0 = dist == m0
    d1 = jnp.where(c0, inf, dist)
    m1 = jnp.min(d1, axis=0, keepdims=True)
    c1 = d1 == m1
    d2 = jnp.where(c1, inf, d1)
    m2 = jnp.min(d2, axis=0, keepdims=True)
    c2 = d2 == m2

    r0 = 1.0 / (m0 + 1e-8)
    r1 = 1.0 / (m1 + 1e-8)
    r2 = 1.0 / (m2 + 1e-8)
    rnorm = r0 + r1 + r2
    w0 = r0 / rnorm
    w1 = r1 / rnorm
    w2 = r2 / rnorm

    zero = jnp.zeros((S, NB), jnp.float32)
    wdT = (jnp.where(c0, w0, zero)
           + jnp.where(c1, w1, zero)
           + jnp.where(c2, w2, zero))

    interpT = jax.lax.dot_general(p2_ref[0], wdT, (((1,), (0,)), ((), ())),
                                  preferred_element_type=jnp.float32)
    z1 = (jax.lax.dot_general(w0i_ref[...], interpT, (((1,), (0,)), ((), ())),
                              preferred_element_type=jnp.float32)
          + jax.lax.dot_general(w0p_ref[...], p1_ref[0], (((1,), (0,)), ((), ())),
                                preferred_element_type=jnp.float32))
    z1_ref[0] = z1

    part = jnp.concatenate(
        [jnp.sum(z1, axis=1, keepdims=True),
         jnp.sum(z1 * z1, axis=1, keepdims=True)], axis=1)
    first = (pl.program_id(0) == 0) & (pl.program_id(1) == 0)

    @pl.when(first)
    def _():
        sums_ref[...] = part

    @pl.when(jnp.logical_not(first))
    def _():
        sums_ref[...] = sums_ref[...] + part


def _k2_body(z1_ref, sums_ref, g_ref, be_ref, w1_ref, z2_ref, sums2_ref,
             *, count):
    s = sums_ref[:, 0:1]
    sq = sums_ref[:, 1:2]
    mu = s / count
    var = sq / count - mu * mu
    inv = g_ref[...] * jax.lax.rsqrt(var + 1e-5)
    shift = be_ref[...] - mu * inv
    h = jnp.maximum(z1_ref[0] * inv + shift, 0.0)
    z2 = jax.lax.dot_general(w1_ref[...], h, (((1,), (0,)), ((), ())),
                             preferred_element_type=jnp.float32)
    z2_ref[0] = z2

    part = jnp.concatenate(
        [jnp.sum(z2, axis=1, keepdims=True),
         jnp.sum(z2 * z2, axis=1, keepdims=True)], axis=1)
    first = (pl.program_id(0) == 0) & (pl.program_id(1) == 0)

    @pl.when(first)
    def _():
        sums2_ref[...] = part

    @pl.when(jnp.logical_not(first))
    def _():
        sums2_ref[...] = sums2_ref[...] + part


def _k3_body(z2_ref, sums_ref, g_ref, be_ref, y_ref, *, count):
    s = sums_ref[:, 0:1]
    sq = sums_ref[:, 1:2]
    mu = s / count
    var = sq / count - mu * mu
    inv = g_ref[...] * jax.lax.rsqrt(var + 1e-5)
    shift = be_ref[...] - mu * inv
    y_ref[0] = jnp.maximum(z2_ref[0] * inv + shift, 0.0)


def kernel(xyz1, xyz2, points1, points2, feat1, feat2,
           W0, b0, g0, be0, W1, b1, g1, be1):
    B, _, N = xyz1.shape
    S = xyz2.shape[2]
    D1 = points1.shape[1]
    D2 = points2.shape[1]
    CF = feat1.shape[2]
    C1 = W0.shape[0]
    C2 = W1.shape[0]
    NB = min(256, N)
    NJ = N // NB
    C = 3 + CF
    CP = ((C + 7) // 8) * 8

    x1t = jnp.transpose(xyz1, (0, 2, 1))
    x2t = jnp.transpose(xyz2, (0, 2, 1))
    q = jnp.concatenate(
        [x1t, feat1, jnp.zeros((B, N, CP - C), jnp.float32)], axis=2)
    kq = jnp.concatenate(
        [x2t, feat2, jnp.zeros((B, S, CP - C), jnp.float32)], axis=2)
    w0p = W0[:, :D1]
    w0i = W0[:, D1:]
    g0c = g0[:, None]
    be0c = be0[:, None]
    g1c = g1[:, None]
    be1c = be1[:, None]

    z1, sums1 = pl.pallas_call(
        functools.partial(_k1_body, S=S, NB=NB),
        grid=(B, NJ),
        in_specs=[
            pl.BlockSpec((1, NB, CP), lambda b, j: (b, j, 0)),
            pl.BlockSpec((1, S, CP), lambda b, j: (b, 0, 0)),
            pl.BlockSpec((1, D2, S), lambda b, j: (b, 0, 0)),
            pl.BlockSpec((1, D1, NB), lambda b, j: (b, 0, j)),
            pl.BlockSpec((C1, D1), lambda b, j: (0, 0)),
            pl.BlockSpec((C1, D2), lambda b, j: (0, 0)),
        ],
        out_specs=[
            pl.BlockSpec((1, C1, NB), lambda b, j: (b, 0, j)),
            pl.BlockSpec((C1, 2), lambda b, j: (0, 0)),
        ],
        out_shape=[
            jax.ShapeDtypeStruct((B, C1, N), jnp.float32),
            jax.ShapeDtypeStruct((C1, 2), jnp.float32),
        ],
    )(q, kq, points2, points1, w0p, w0i)

    count = float(B * N)
    z2, sums2 = pl.pallas_call(
        functools.partial(_k2_body, count=count),
        grid=(B, NJ),
        in_specs=[
            pl.BlockSpec((1, C1, NB), lambda b, j: (b, 0, j)),
            pl.BlockSpec((C1, 2), lambda b, j: (0, 0)),
            pl.BlockSpec((C1, 1), lambda b, j: (0, 0)),
            pl.BlockSpec((C1, 1), lambda b, j: (0, 0)),
            pl.BlockSpec((C2, C1), lambda b, j: (0, 0)),
        ],
        out_specs=[
            pl.BlockSpec((1, C2, NB), lambda b, j: (b, 0, j)),
            pl.BlockSpec((C2, 2), lambda b, j: (0, 0)),
        ],
        out_shape=[
            jax.ShapeDtypeStruct((B, C2, N), jnp.float32),
            jax.ShapeDtypeStruct((C2, 2), jnp.float32),
        ],
    )(z1, sums1, g0c, be0c, W1)

    y = pl.pallas_call(
        functools.partial(_k3_body, count=count),
        grid=(B, NJ),
        in_specs=[
            pl.BlockSpec((1, C2, NB), lambda b, j: (b, 0, j)),
            pl.BlockSpec((C2, 2), lambda b, j: (0, 0)),
            pl.BlockSpec((C2, 1), lambda b, j: (0, 0)),
            pl.BlockSpec((C2, 1), lambda b, j: (0, 0)),
        ],
        out_specs=pl.BlockSpec((1, C2, NB), lambda b, j: (b, 0, j)),
        out_shape=jax.ShapeDtypeStruct((B, C2, N), jnp.float32),
    )(z2, sums2, g1c, be1c)

    return y
